# baseline (speedup 1.0000x reference)
import functools

import jax
import jax.numpy as jnp
from jax import lax
from jax.experimental import pallas as pl
from jax.experimental.pallas import tpu as pltpu
from jax.experimental.pallas import tpu_sc as plsc

N_ROWS = 1000000
EMBED_DIM = 64
BATCH = 16384

NC, NS, L = 2, 16, 16
NW = NC * NS
N_UBLK = (N_ROWS + 127) // 128
RANGES = NW // 2
RANGE = (N_UBLK + RANGES - 1) // RANGES
SCAN_CHUNKS = BATCH // L
SLOTS_PAD = 544
RB = 16
RING = 9

B_PER_W = BATCH // NW
HALF = 256


def _zero_i32(ref, n):
    z = jnp.zeros((L,), jnp.int32)
    for k in range(n // L):
        ref[pl.ds(k * L, L)] = z


def _sc1(v):
    return v[0]


def _extract_half(nat_hbm, ids_hbm, out3_hbm, rid,
                  ids_v, sel_k, ord_k,
                  marks, counts, starts, cursors, slots_list,
                  buf3, rowbuf2, poslist2, sem0, sem1):
    lo = rid * RANGE
    lane16 = lax.iota(jnp.int32, L)

    pltpu.sync_copy(ids_hbm, ids_v)

    _zero_i32(marks, SLOTS_PAD)
    _zero_i32(counts, SLOTS_PAD)

    ones16 = jnp.ones((L,), jnp.int32)

    def scan(k, n):
        idv = ids_v[pl.ds(k * L, L)]
        slot = lax.shift_right_logical(idv, 7) - lo
        m = (slot >= 0) & (slot < RANGE)
        key = (lax.shift_left(slot, 21) | lax.shift_left(idv & 127, 14)
               | (lane16 + k * L))
        plsc.store_compressed(sel_k.at[pl.ds(n, L)], key, mask=m)
        plsc.store_scatter(marks, [slot], ones16, mask=m)
        plsc.addupdate_scatter(counts, [slot], ones16, mask=m)
        return n + _sc1(plsc.all_reduce_population_count(m))

    n = lax.fori_loop(0, SCAN_CHUNKS, scan, 0)

    def psum(k, s):
        c = counts[pl.ds(k * L, L)]
        cs = plsc.cumsum(c)
        starts[pl.ds(k * L, L)] = cs - c + s
        cursors[pl.ds(k * L, L)] = cs - c + s
        return s + cs[L - 1]

    lax.fori_loop(0, SLOTS_PAD // L, psum, 0)

    def reorder(j, c):
        key = _sc1(sel_k[pl.ds(j, L)])
        slot = lax.shift_right_logical(key, 21)
        dst = _sc1(cursors[pl.ds(slot, L)])
        m0 = lane16 == 0
        plsc.store_scatter(ord_k, [jnp.full((L,), dst, jnp.int32)],
                           jnp.full((L,), key, jnp.int32), mask=m0)
        plsc.store_scatter(cursors, [jnp.full((L,), slot, jnp.int32)],
                           jnp.full((L,), dst + 1, jnp.int32), mask=m0)
        return c

    lax.fori_loop(0, n, reorder, 0)

    def compact(k, u):
        m = marks[pl.ds(k * L, L)] > 0
        plsc.store_compressed(slots_list.at[pl.ds(u, L)],
                              lane16 + k * L, mask=m)
        return u + _sc1(plsc.all_reduce_population_count(m))

    u = lax.fori_loop(0, SLOTS_PAD // L, compact, 0)

    neg1 = jnp.full((L,), -1, jnp.int32)
    poslist2[0, pl.ds(0, L)] = neg1
    poslist2[1, pl.ds(0, L)] = neg1

    def issue(k):
        slot = _sc1(slots_list[pl.ds(k, L)])
        pltpu.async_copy(
            nat_hbm.at[:, pl.ds((lo + slot) * 128, 128)],
            buf3.at[lax.rem(k, RING)], sem0)

    def flush_wait():
        pltpu.make_async_copy(
            rowbuf2.at[0], out3_hbm.at[pl.ds(0, RB), :], sem1).wait()

    @pl.when(u > 0)
    def _fetch_extract():
        for p in range(RING - 1):
            issue(lax.min(p, u - 1))

        def body(k, carry):
            mcount, fp, fl = carry
            par = lax.rem(k, RING)
            pltpu.make_async_copy(
                nat_hbm.at[:, pl.ds(0, 128)], buf3.at[par], sem0).wait()
            issue(lax.min(k + RING - 1, u - 1))

            slot = _sc1(slots_list[pl.ds(k, L)])
            st = _sc1(starts[pl.ds(slot, L)])
            cnt = _sc1(counts[pl.ds(slot, L)])

            def user(j, ucarry):
                mc, fp, fl = ucarry
                key = _sc1(ord_k[pl.ds(j, L)])
                lnv = jnp.full((L,), lax.shift_right_logical(key, 14) & 127,
                               jnp.int32)
                parv = jnp.full((L,), par, jnp.int32)
                for c in range(EMBED_DIM // L):
                    vals = plsc.load_gather(
                        buf3, [parv, c * L + lane16, lnv])
                    rowbuf2[fp, mc, pl.ds(c * L, L)] = vals
                m0 = lane16 == 0
                plsc.store_scatter(
                    poslist2, [jnp.full((L,), fp, jnp.int32),
                               jnp.full((L,), mc, jnp.int32)],
                    jnp.full((L,), key & 16383, jnp.int32), mask=m0)
                mc = mc + 1

                def do_flush(args):
                    fp, fl = args
                    pltpu.async_copy(
                        rowbuf2.at[fp],
                        out3_hbm.at[plsc.Indices(poslist2.at[fp],
                                                 ignored_value=-1)],
                        sem1)
                    pl.when(fl >= 1)(flush_wait)
                    nfp = 1 - fp
                    poslist2[nfp, pl.ds(0, L)] = neg1
                    return nfp, fl + 1

                fp, fl = lax.cond(mc == RB, do_flush, lambda a: a, (fp, fl))
                return lax.rem(mc, RB), fp, fl

            mc, fp, fl = lax.fori_loop(st, st + cnt, user,
                                       (mcount, fp, fl))
            return (mc, fp, fl)

        mc, fp, fl = lax.fori_loop(0, u, body, (0, 0, 0))
        for p in range(RING - 1):
            pltpu.make_async_copy(
                nat_hbm.at[:, pl.ds(0, 128)], buf3.at[p], sem0).wait()
        pltpu.sync_copy(
            rowbuf2.at[fp],
            out3_hbm.at[plsc.Indices(poslist2.at[fp], ignored_value=-1)])
        pl.when(fl >= 1)(flush_wait)


def _phase_a_body(user_nat, item_nat, uid_hbm, iid_hbm, u3_hbm, v3_hbm,
                  ids_v, sel_k, ord_k,
                  marks, counts, starts, cursors, slots_list,
                  buf3, rowbuf2, poslist2, sem0, sem1):
    wid = lax.axis_index("s") * NC + lax.axis_index("c")
    half = wid // RANGES
    rid = lax.rem(wid, RANGES)
    scratch = (ids_v, sel_k, ord_k,
               marks, counts, starts, cursors, slots_list,
               buf3, rowbuf2, poslist2, sem0, sem1)

    @pl.when(half == 0)
    def _():
        _extract_half(user_nat, uid_hbm, u3_hbm, rid, *scratch)

    @pl.when(half == 1)
    def _():
        _extract_half(item_nat, iid_hbm, v3_hbm, rid, *scratch)


@functools.partial(
    pl.kernel,
    out_type=(jax.ShapeDtypeStruct((BATCH, 128), jnp.float32),
              jax.ShapeDtypeStruct((BATCH, 128), jnp.float32)),
    mesh=plsc.VectorSubcoreMesh(core_axis_name="c", subcore_axis_name="s",
                                num_cores=NC, num_subcores=NS),
    compiler_params=pltpu.CompilerParams(needs_layout_passes=False,
                                         use_tc_tiling_on_sc=True),
    scratch_types=[
        pltpu.VMEM((BATCH,), jnp.int32),
        pltpu.VMEM((BATCH + L,), jnp.int32),
        pltpu.VMEM((BATCH + L,), jnp.int32),
        pltpu.VMEM((SLOTS_PAD,), jnp.int32),
        pltpu.VMEM((SLOTS_PAD,), jnp.int32),
        pltpu.VMEM((SLOTS_PAD,), jnp.int32),
        pltpu.VMEM((SLOTS_PAD,), jnp.int32),
        pltpu.VMEM((SLOTS_PAD,), jnp.int32),
        pltpu.VMEM((RING, EMBED_DIM, 128), jnp.float32),
        pltpu.VMEM((2, RB, 128), jnp.float32),
        pltpu.VMEM((2, RB), jnp.int32),
        pltpu.SemaphoreType.DMA,
        pltpu.SemaphoreType.DMA,
    ],
)
def _phase_a(*args):
    _phase_a_body(*args)


def _phase_b_body(u3_hbm, v3_hbm, out_hbm, ubuf, ibuf, out_v, tr_v, sem):
    wid = lax.axis_index("s") * NC + lax.axis_index("c")
    base = wid * B_PER_W
    lane = lax.iota(jnp.int32, L)
    col_addr = lane * (L + 1)

    def half_step(h, carry):
        hbase = base + h * HALF
        cp_u = pltpu.async_copy(u3_hbm.at[pl.ds(hbase, HALF), :], ubuf, sem)
        cp_u.wait()
        cp_i = pltpu.async_copy(v3_hbm.at[pl.ds(hbase, HALF), :], ibuf, sem)
        cp_i.wait()

        def group(g, c2):
            row0 = g * L
            for r in range(L):
                s = ubuf[row0 + r, pl.ds(0, L)] * ibuf[row0 + r, pl.ds(0, L)]
                for d in range(1, EMBED_DIM // L):
                    s = s + (ubuf[row0 + r, pl.ds(d * L, L)]
                             * ibuf[row0 + r, pl.ds(d * L, L)])
                plsc.store_scatter(tr_v, [col_addr + r], s)
            acc = tr_v[pl.ds(0, L)]
            for l in range(1, L):
                acc = acc + tr_v[pl.ds(l * (L + 1), L)]
            out_v[pl.ds(h * HALF + row0, L)] = acc
            return c2

        lax.fori_loop(0, HALF // L, group, 0)
        return carry

    lax.fori_loop(0, B_PER_W // HALF, half_step, 0)
    pltpu.sync_copy(out_v, out_hbm.at[pl.ds(base, B_PER_W)])


@functools.partial(
    pl.kernel,
    out_type=jax.ShapeDtypeStruct((BATCH,), jnp.float32),
    mesh=plsc.VectorSubcoreMesh(core_axis_name="c", subcore_axis_name="s",
                                num_cores=NC, num_subcores=NS),
    compiler_params=pltpu.CompilerParams(needs_layout_passes=False,
                                         use_tc_tiling_on_sc=True),
    scratch_types=[
        pltpu.VMEM((HALF, 128), jnp.float32),
        pltpu.VMEM((HALF, 128), jnp.float32),
        pltpu.VMEM((B_PER_W,), jnp.float32),
        pltpu.VMEM((L * (L + 1),), jnp.float32),
        pltpu.SemaphoreType.DMA,
    ],
)
def _phase_b(*args):
    _phase_b_body(*args)


def kernel(uid, iid, user_table, item_table):
    u3, v3 = _phase_a(user_table.T, item_table.T, uid, iid)
    return _phase_b(u3, v3)


# speedup vs baseline: 1.0057x; 1.0057x over previous
import functools

import jax
import jax.numpy as jnp
from jax import lax
from jax.experimental import pallas as pl
from jax.experimental.pallas import tpu as pltpu
from jax.experimental.pallas import tpu_sc as plsc

N_ROWS = 1000000
EMBED_DIM = 64
BATCH = 16384

NC, NS, L = 2, 16, 16
NW = NC * NS
N_UBLK = (N_ROWS + 127) // 128
RANGES = NW // 2
RANGE = (N_UBLK + RANGES - 1) // RANGES
SCAN_CHUNKS = BATCH // L
SLOTS_PAD = 544
RB = 16
RING = 9

B_PER_W = BATCH // NW
HALF = 256


def _zero_i32(ref, n):
    z = jnp.zeros((L,), jnp.int32)
    for k in range(n // L):
        ref[pl.ds(k * L, L)] = z


def _sc1(v):
    return v[0]


def _extract_half(nat_hbm, ids_hbm, out3_hbm, rid,
                  ids_v, sel_k, ord_k,
                  marks, counts, starts, cursors, slots_list,
                  buf3, rowbuf2, poslist2, sem0, sem1):
    lo = rid * RANGE
    lane16 = lax.iota(jnp.int32, L)

    pltpu.sync_copy(ids_hbm, ids_v)

    _zero_i32(marks, SLOTS_PAD)
    _zero_i32(counts, SLOTS_PAD)

    ones16 = jnp.ones((L,), jnp.int32)

    def scan(k, n):
        idv = ids_v[pl.ds(k * L, L)]
        slot = lax.shift_right_logical(idv, 7) - lo
        m = (slot >= 0) & (slot < RANGE)
        key = (lax.shift_left(slot, 21) | lax.shift_left(idv & 127, 14)
               | (lane16 + k * L))
        plsc.store_compressed(sel_k.at[pl.ds(n, L)], key, mask=m)
        plsc.store_scatter(marks, [slot], ones16, mask=m)
        plsc.addupdate_scatter(counts, [slot], ones16, mask=m)
        return n + _sc1(plsc.all_reduce_population_count(m))

    n = lax.fori_loop(0, SCAN_CHUNKS, scan, 0)

    def psum(k, s):
        c = counts[pl.ds(k * L, L)]
        cs = plsc.cumsum(c)
        starts[pl.ds(k * L, L)] = cs - c + s
        cursors[pl.ds(k * L, L)] = cs - c + s
        return s + cs[L - 1]

    lax.fori_loop(0, SLOTS_PAD // L, psum, 0)

    def reorder(j, c):
        key = _sc1(sel_k[pl.ds(j, L)])
        slot = lax.shift_right_logical(key, 21)
        dst = _sc1(cursors[pl.ds(slot, L)])
        m0 = lane16 == 0
        plsc.store_scatter(ord_k, [jnp.full((L,), dst, jnp.int32)],
                           jnp.full((L,), key, jnp.int32), mask=m0)
        plsc.store_scatter(cursors, [jnp.full((L,), slot, jnp.int32)],
                           jnp.full((L,), dst + 1, jnp.int32), mask=m0)
        return c

    lax.fori_loop(0, n, reorder, 0)

    def compact(k, u):
        m = marks[pl.ds(k * L, L)] > 0
        plsc.store_compressed(slots_list.at[pl.ds(u, L)],
                              lane16 + k * L, mask=m)
        return u + _sc1(plsc.all_reduce_population_count(m))

    u = lax.fori_loop(0, SLOTS_PAD // L, compact, 0)

    neg1 = jnp.full((L,), -1, jnp.int32)
    poslist2[0, pl.ds(0, L)] = neg1
    poslist2[1, pl.ds(0, L)] = neg1

    def issue(k):
        slot = _sc1(slots_list[pl.ds(k, L)])
        pltpu.async_copy(
            nat_hbm.at[:, pl.ds((lo + slot) * 128, 128)],
            buf3.at[lax.rem(k, RING)], sem0)

    def flush_wait():
        pltpu.make_async_copy(
            rowbuf2.at[0], out3_hbm.at[pl.ds(0, RB), :], sem1).wait()

    @pl.when(u > 0)
    def _fetch_extract():
        for p in range(RING - 1):
            issue(lax.min(p, u - 1))

        def body(k, carry):
            mcount, fp, fl = carry
            par = lax.rem(k, RING)
            pltpu.make_async_copy(
                nat_hbm.at[:, pl.ds(0, 128)], buf3.at[par], sem0).wait()
            issue(lax.min(k + RING - 1, u - 1))

            slot = _sc1(slots_list[pl.ds(k, L)])
            st = _sc1(starts[pl.ds(slot, L)])
            cnt = _sc1(counts[pl.ds(slot, L)])

            def user(j, ucarry):
                mc, fp, fl = ucarry
                key = _sc1(ord_k[pl.ds(j, L)])
                lnv = jnp.full((L,), lax.shift_right_logical(key, 14) & 127,
                               jnp.int32)
                parv = jnp.full((L,), par, jnp.int32)
                for c in range(EMBED_DIM // L):
                    vals = plsc.load_gather(
                        buf3, [parv, c * L + lane16, lnv])
                    rowbuf2[fp, mc, pl.ds(c * L, L)] = vals
                m0 = lane16 == 0
                plsc.store_scatter(
                    poslist2, [jnp.full((L,), fp, jnp.int32),
                               jnp.full((L,), mc, jnp.int32)],
                    jnp.full((L,), key & 16383, jnp.int32), mask=m0)
                mc = mc + 1

                def do_flush(args):
                    fp, fl = args
                    pltpu.async_copy(
                        rowbuf2.at[fp],
                        out3_hbm.at[plsc.Indices(poslist2.at[fp],
                                                 ignored_value=-1)],
                        sem1)
                    pl.when(fl >= 1)(flush_wait)
                    nfp = 1 - fp
                    poslist2[nfp, pl.ds(0, L)] = neg1
                    return nfp, fl + 1

                fp, fl = lax.cond(mc == RB, do_flush, lambda a: a, (fp, fl))
                return lax.rem(mc, RB), fp, fl

            mc, fp, fl = lax.fori_loop(st, st + cnt, user,
                                       (mcount, fp, fl))
            return (mc, fp, fl)

        mc, fp, fl = lax.fori_loop(0, u, body, (0, 0, 0))
        for p in range(RING - 1):
            pltpu.make_async_copy(
                nat_hbm.at[:, pl.ds(0, 128)], buf3.at[p], sem0).wait()
        pltpu.sync_copy(
            rowbuf2.at[fp],
            out3_hbm.at[plsc.Indices(poslist2.at[fp], ignored_value=-1)])
        pl.when(fl >= 1)(flush_wait)


def _phase_a_body(user_nat, item_nat, uid_hbm, iid_hbm, u3_hbm, v3_hbm,
                  ids_v, sel_k, ord_k,
                  marks, counts, starts, cursors, slots_list,
                  buf3, rowbuf2, poslist2, sem0, sem1):
    wid = lax.axis_index("s") * NC + lax.axis_index("c")
    half = wid // RANGES
    rid = lax.rem(wid, RANGES)
    scratch = (ids_v, sel_k, ord_k,
               marks, counts, starts, cursors, slots_list,
               buf3, rowbuf2, poslist2, sem0, sem1)

    @pl.when(half == 0)
    def _():
        _extract_half(user_nat, uid_hbm, u3_hbm, rid, *scratch)

    @pl.when(half == 1)
    def _():
        _extract_half(item_nat, iid_hbm, v3_hbm, rid, *scratch)


@functools.partial(
    pl.kernel,
    out_type=(jax.ShapeDtypeStruct((BATCH, 128), jnp.float32),
              jax.ShapeDtypeStruct((BATCH, 128), jnp.float32)),
    mesh=plsc.VectorSubcoreMesh(core_axis_name="c", subcore_axis_name="s",
                                num_cores=NC, num_subcores=NS),
    compiler_params=pltpu.CompilerParams(needs_layout_passes=False,
                                         use_tc_tiling_on_sc=True),
    scratch_types=[
        pltpu.VMEM((BATCH,), jnp.int32),
        pltpu.VMEM((BATCH + L,), jnp.int32),
        pltpu.VMEM((BATCH + L,), jnp.int32),
        pltpu.VMEM((SLOTS_PAD,), jnp.int32),
        pltpu.VMEM((SLOTS_PAD,), jnp.int32),
        pltpu.VMEM((SLOTS_PAD,), jnp.int32),
        pltpu.VMEM((SLOTS_PAD,), jnp.int32),
        pltpu.VMEM((SLOTS_PAD,), jnp.int32),
        pltpu.VMEM((RING, EMBED_DIM, 128), jnp.float32),
        pltpu.VMEM((2, RB, 128), jnp.float32),
        pltpu.VMEM((2, RB), jnp.int32),
        pltpu.SemaphoreType.DMA,
        pltpu.SemaphoreType.DMA,
    ],
)
def _phase_a(*args):
    _phase_a_body(*args)


def _phase_b_body(u3_hbm, v3_hbm, out_hbm, ubuf, ibuf, out_v, tr_v, sem):
    wid = lax.axis_index("s") * NC + lax.axis_index("c")
    base = wid * B_PER_W
    lane = lax.iota(jnp.int32, L)
    col_addr = lane * (L + 1)

    def half_step(h, carry):
        hbase = base + h * HALF
        cp_u = pltpu.async_copy(u3_hbm.at[pl.ds(hbase, HALF), :], ubuf, sem)
        cp_i = pltpu.async_copy(v3_hbm.at[pl.ds(hbase, HALF), :], ibuf, sem)
        cp_u.wait()
        cp_i.wait()

        def group(g, c2):
            row0 = g * L
            for r in range(L):
                s = ubuf[row0 + r, pl.ds(0, L)] * ibuf[row0 + r, pl.ds(0, L)]
                for d in range(1, EMBED_DIM // L):
                    s = s + (ubuf[row0 + r, pl.ds(d * L, L)]
                             * ibuf[row0 + r, pl.ds(d * L, L)])
                plsc.store_scatter(tr_v, [col_addr + r], s)
            acc = tr_v[pl.ds(0, L)]
            for l in range(1, L):
                acc = acc + tr_v[pl.ds(l * (L + 1), L)]
            out_v[pl.ds(h * HALF + row0, L)] = acc
            return c2

        lax.fori_loop(0, HALF // L, group, 0)
        return carry

    lax.fori_loop(0, B_PER_W // HALF, half_step, 0)
    pltpu.sync_copy(out_v, out_hbm.at[pl.ds(base, B_PER_W)])


@functools.partial(
    pl.kernel,
    out_type=jax.ShapeDtypeStruct((BATCH,), jnp.float32),
    mesh=plsc.VectorSubcoreMesh(core_axis_name="c", subcore_axis_name="s",
                                num_cores=NC, num_subcores=NS),
    compiler_params=pltpu.CompilerParams(needs_layout_passes=False,
                                         use_tc_tiling_on_sc=True),
    scratch_types=[
        pltpu.VMEM((HALF, 128), jnp.float32),
        pltpu.VMEM((HALF, 128), jnp.float32),
        pltpu.VMEM((B_PER_W,), jnp.float32),
        pltpu.VMEM((L * (L + 1),), jnp.float32),
        pltpu.SemaphoreType.DMA,
    ],
)
def _phase_b(*args):
    _phase_b_body(*args)


def kernel(uid, iid, user_table, item_table):
    u3, v3 = _phase_a(user_table.T, item_table.T, uid, iid)
    return _phase_b(u3, v3)
